# baseline (speedup 1.0000x reference)
import functools

import jax
import jax.numpy as jnp
from jax import lax
from jax.experimental import pallas as pl
from jax.experimental.pallas import tpu as pltpu
from jax.experimental.pallas import tpu_sc as plsc

LANES = 16


@functools.lru_cache(maxsize=None)
def _build(bt: int, c: int, v: int, d: int):
    info = plsc.get_sparse_core_info()
    nc, ns = info.num_cores, info.num_subcores
    nw = nc * ns
    chunk = 16
    assert bt % (nw * chunk) == 0
    pos_per_w = bt // nw
    nchunk = pos_per_w // chunk
    assert nchunk % 2 == 0 and nchunk >= 6
    rows = chunk * c
    assert rows % LANES == 0 and d % LANES == 0 and LANES % c == 0

    mesh = plsc.VectorSubcoreMesh(core_axis_name="c", subcore_axis_name="s")

    def body(tab, tok, scl, out, tok_all, gb0, gb1, ob0, ob1,
             scale_v, gsem0, gsem1, osem0, osem1):
        wid = lax.axis_index("s") * nc + lax.axis_index("c")
        base = wid * pos_per_w

        pltpu.sync_copy(scl, scale_v)
        pltpu.sync_copy(tok.at[pl.ds(base * c, pos_per_w * c)], tok_all)
        svec = [scale_v[pl.ds(i * LANES, LANES)] for i in range(c)]
        offpat = (lax.iota(jnp.int32, LANES) % c) * v

        @plsc.parallel_loop(0, pos_per_w * c // LANES, unroll=4)
        def _mkidx(k):
            sl = pl.ds(k * LANES, LANES)
            tok_all[sl] = tok_all[sl] + offpat

        gbs = (gb0, gb1)
        obs = (ob0, ob1)
        gsems = (gsem0, gsem1)
        osems = (osem0, osem1)

        def issue(g, b):
            idx = tok_all.at[pl.ds(g * rows, rows)]
            pltpu.make_async_copy(tab.at[idx], gbs[b], gsems[b]).start()

        def accumulate(b):
            gb, ob = gbs[b], obs[b]

            @plsc.parallel_loop(0, chunk)
            def _pos(p):
                r0 = p * c

                @plsc.parallel_loop(0, d // LANES, unroll=2)
                def _blk(di):
                    sl = pl.ds(di * LANES, LANES)
                    terms = [gb[r0 + i, sl] * svec[i] for i in range(c)]
                    while len(terms) > 1:
                        nxt = [terms[j] + terms[j + 1]
                               for j in range(0, len(terms) - 1, 2)]
                        if len(terms) % 2:
                            nxt[-1] = nxt[-1] + terms[-1]
                        terms = nxt
                    ob[p, sl] = terms[0]

        def out_copy(g, b):
            return pltpu.make_async_copy(
                obs[b], out.at[pl.ds(base + g * chunk, chunk)], osems[b])

        def consume(g, b, drain):
            idx = tok_all.at[pl.ds(g * rows, rows)]
            pltpu.make_async_copy(tab.at[idx], gbs[b], gsems[b]).wait()
            if drain:
                out_copy(g - 2, b).wait()
            accumulate(b)
            out_copy(g, b).start()

        issue(0, 0)
        issue(1, 1)
        consume(0, 0, drain=False)
        issue(2, 0)
        consume(1, 1, drain=False)
        issue(3, 1)

        def outer(i, carry):
            g0 = 2 * i
            consume(g0, 0, drain=True)
            issue(g0 + 2, 0)
            consume(g0 + 1, 1, drain=True)
            issue(g0 + 3, 1)
            return carry

        lax.fori_loop(1, nchunk // 2 - 1, outer, 0)
        consume(nchunk - 2, 0, drain=True)
        consume(nchunk - 1, 1, drain=True)
        out_copy(nchunk - 2, 0).wait()
        out_copy(nchunk - 1, 1).wait()

    return pl.kernel(
        body,
        out_type=jax.ShapeDtypeStruct((bt, d), jnp.float32),
        mesh=mesh,
        scratch_types=[
            pltpu.VMEM((pos_per_w * c,), jnp.int32),
            pltpu.VMEM((rows, d), jnp.float32),
            pltpu.VMEM((rows, d), jnp.float32),
            pltpu.VMEM((chunk, d), jnp.float32),
            pltpu.VMEM((chunk, d), jnp.float32),
            pltpu.VMEM((c * LANES,), jnp.float32),
            pltpu.SemaphoreType.DMA,
            pltpu.SemaphoreType.DMA,
            pltpu.SemaphoreType.DMA,
            pltpu.SemaphoreType.DMA,
        ],
    )


def kernel(tokens, tables, level_scale):
    b, t, c = tokens.shape
    _, v, d = tables.shape
    tok_flat = tokens.astype(jnp.int32).reshape(b * t * c)
    tab_flat = tables.reshape(c * v, d)
    scl = jnp.repeat(level_scale.astype(jnp.float32), LANES)
    out = _build(b * t, c, v, d)(tab_flat, tok_flat, scl)
    return out.reshape(b, t, d)
